# baseline (speedup 1.0000x reference)
import functools

import jax
import jax.numpy as jnp
from jax import lax
from jax.experimental import pallas as pl
from jax.experimental.pallas import tpu as pltpu
from jax.experimental.pallas import tpu_sc as plsc

_NR = 2
_TW = 512


def _transpose_table(tab_t, v_main):
    E, V = tab_t.shape

    def body(in_ref, out_ref):
        u = in_ref[...].reshape(E, _TW // 2, 2)
        w = jnp.transpose(u, (1, 2, 0))
        out_ref[...] = w.reshape(_TW // 2, 2 * E)

    return pl.pallas_call(
        body,
        grid=(v_main // _TW,),
        in_specs=[pl.BlockSpec((E, _TW), lambda i: (0, i))],
        out_specs=pl.BlockSpec((_TW // 2, 2 * E), lambda i: (i, 0)),
        out_shape=jax.ShapeDtypeStruct((v_main // 2, 2 * E), jnp.float32),
    )(tab_t)


def _gather_mean(xa, xb, table, fixup):
    B, S = xa.shape
    Vm, E = table.shape
    NC, NS = 2, 16
    NW = NC * NS
    rows_per_w = B // NW
    nvec = E // 16
    s_a = 128
    s_b = S - s_a
    mesh = plsc.VectorSubcoreMesh(core_axis_name="c", subcore_axis_name="s")

    @functools.partial(
        pl.kernel,
        mesh=mesh,
        out_type=jax.ShapeDtypeStruct((B, E), jnp.float32),
        compiler_params=pltpu.CompilerParams(use_tc_tiling_on_sc=False),
        scratch_types=[
            pltpu.VMEM((rows_per_w, S), jnp.int32),
            pltpu.VMEM((rows_per_w, S), jnp.int32),
            pltpu.VMEM((_NR, S, E), jnp.float32),
            pltpu.VMEM((_NR, S, E), jnp.float32),
            pltpu.VMEM((rows_per_w, E), jnp.float32),
        ]
        + [pltpu.SemaphoreType.DMA] * _NR,
    )
    def k(xa_hbm, xb_hbm, tab_hbm, fix_hbm, out_hbm,
          idxa_v, idxb_v, bufsa, bufsb, out_v, *sems):
        wid = lax.axis_index("s") * NC + lax.axis_index("c")
        rbase = wid * rows_per_w
        pltpu.sync_copy(xa_hbm.at[pl.ds(rbase, rows_per_w)], idxa_v)
        pltpu.sync_copy(xb_hbm.at[pl.ds(rbase, rows_per_w)], idxb_v)

        def one(src, idx_v, bufs, r, n, lo, ln):
            return pltpu.make_async_copy(
                src.at[idx_v.at[r, pl.ds(lo, ln)]],
                bufs.at[n, pl.ds(lo, ln)],
                sems[n],
            )

        def fire(r, n):
            one(tab_hbm, idxa_v, bufsa, r, n, 0, s_a).start()
            one(tab_hbm, idxa_v, bufsa, r, n, s_a, s_b).start()
            one(fix_hbm, idxb_v, bufsb, r, n, 0, s_a).start()
            one(fix_hbm, idxb_v, bufsb, r, n, s_a, s_b).start()

        def drain(n):
            one(tab_hbm, idxa_v, bufsa, 0, n, 0, s_a).wait()
            one(tab_hbm, idxa_v, bufsa, 0, n, s_a, s_b).wait()
            one(fix_hbm, idxb_v, bufsb, 0, n, 0, s_a).wait()
            one(fix_hbm, idxb_v, bufsb, 0, n, s_a, s_b).wait()

        def accum(buf, init):
            unroll = 8

            def body(t, a):
                base = t * unroll
                for i in range(unroll):
                    a = tuple(
                        a[q] + buf[base + i, pl.ds(16 * q, 16)]
                        for q in range(nvec)
                    )
                return a

            return lax.fori_loop(0, S // unroll, body, init)

        scale = jnp.float32(1.0 / S)
        zero = jnp.zeros((16,), jnp.float32)

        def do_row(r, n, do_fire):
            drain(n)
            accs = accum(bufsa.at[n], (zero,) * nvec)
            accs = accum(bufsb.at[n], accs)
            if do_fire:
                fire(r + _NR, n)
            for q in range(nvec):
                out_v[r, pl.ds(16 * q, 16)] = accs[q] * scale

        for n in range(_NR):
            fire(n, n)

        def loop_body(g, _):
            for n in range(_NR):
                do_row(_NR * g + n, n, True)
            return 0

        lax.fori_loop(0, rows_per_w // _NR - 1, loop_body, 0)
        for n in range(_NR):
            do_row(rows_per_w - _NR + n, n, False)

        pltpu.sync_copy(out_v, out_hbm.at[pl.ds(rbase, rows_per_w)])

    return k(xa, xb, table, fixup)


@jax.jit
def _embed_mean(x, embedding_weight):
    V, E = embedding_weight.shape
    v_main = (V // 128) * 128
    pairs = _transpose_table(embedding_weight.T, v_main)
    table = pairs.reshape(v_main, E)
    fixup = jnp.concatenate(
        [
            jnp.zeros((1, E), jnp.float32),
            embedding_weight[v_main:] - embedding_weight[v_main - 1],
        ],
        axis=0,
    )
    xa = jnp.minimum(x, v_main - 1)
    xb = jnp.maximum(x - (v_main - 1), 0)
    return _gather_mean(xa, xb, table, fixup)


def kernel(x, embedding_weight):
    return _embed_mean(x, embedding_weight)


# speedup vs baseline: 1.5295x; 1.5295x over previous
import functools

import jax
import jax.numpy as jnp
from jax import lax
from jax.experimental import pallas as pl
from jax.experimental.pallas import tpu as pltpu
from jax.experimental.pallas import tpu_sc as plsc

_NR = 2
_TW = 1536


def _transpose_table(tab_t, v_main):
    E, V = tab_t.shape
    ht = _TW // 2

    def body(in_ref, out_ref):
        t = jnp.transpose(in_ref[...])
        out_ref[:, 0:E] = t[0:ht]
        out_ref[:, E:2 * E] = t[ht:_TW]

    return pl.pallas_call(
        body,
        grid=(v_main // _TW,),
        in_specs=[pl.BlockSpec((E, _TW), lambda i: (0, i))],
        out_specs=pl.BlockSpec((ht, 2 * E), lambda i: (i, 0)),
        out_shape=jax.ShapeDtypeStruct((v_main // 2, 2 * E), jnp.float32),
    )(tab_t)


def _remap(v):
    r = v % _TW
    return v - r + jnp.where(r < _TW // 2, 2 * r, 2 * r - (_TW - 1))


def _gather_mean(xa, xb, table, fixup):
    B, S = xa.shape
    Vm, E = table.shape
    NC, NS = 2, 16
    NW = NC * NS
    rows_per_w = B // NW
    nvec = E // 16
    s_a = 128
    s_b = S - s_a
    mesh = plsc.VectorSubcoreMesh(core_axis_name="c", subcore_axis_name="s")

    @functools.partial(
        pl.kernel,
        mesh=mesh,
        out_type=jax.ShapeDtypeStruct((B, E), jnp.float32),
        compiler_params=pltpu.CompilerParams(use_tc_tiling_on_sc=False),
        scratch_types=[
            pltpu.VMEM((rows_per_w, S), jnp.int32),
            pltpu.VMEM((rows_per_w, S), jnp.int32),
            pltpu.VMEM((_NR, S, E), jnp.float32),
            pltpu.VMEM((_NR, S, E), jnp.float32),
            pltpu.VMEM((rows_per_w, E), jnp.float32),
        ]
        + [pltpu.SemaphoreType.DMA] * _NR,
    )
    def k(xa_hbm, xb_hbm, tab_hbm, fix_hbm, out_hbm,
          idxa_v, idxb_v, bufsa, bufsb, out_v, *sems):
        wid = lax.axis_index("s") * NC + lax.axis_index("c")
        rbase = wid * rows_per_w
        pltpu.sync_copy(xa_hbm.at[pl.ds(rbase, rows_per_w)], idxa_v)
        pltpu.sync_copy(xb_hbm.at[pl.ds(rbase, rows_per_w)], idxb_v)

        def one(src, idx_v, bufs, r, n, lo, ln):
            return pltpu.make_async_copy(
                src.at[idx_v.at[r, pl.ds(lo, ln)]],
                bufs.at[n, pl.ds(lo, ln)],
                sems[n],
            )

        def fire(r, n):
            one(tab_hbm, idxa_v, bufsa, r, n, 0, s_a).start()
            one(tab_hbm, idxa_v, bufsa, r, n, s_a, s_b).start()
            one(fix_hbm, idxb_v, bufsb, r, n, 0, s_a).start()
            one(fix_hbm, idxb_v, bufsb, r, n, s_a, s_b).start()

        def drain(n):
            one(tab_hbm, idxa_v, bufsa, 0, n, 0, s_a).wait()
            one(tab_hbm, idxa_v, bufsa, 0, n, s_a, s_b).wait()
            one(fix_hbm, idxb_v, bufsb, 0, n, 0, s_a).wait()
            one(fix_hbm, idxb_v, bufsb, 0, n, s_a, s_b).wait()

        def accum(buf, init):
            unroll = 8

            def body(t, a):
                base = t * unroll
                for i in range(unroll):
                    a = tuple(
                        a[q] + buf[base + i, pl.ds(16 * q, 16)]
                        for q in range(nvec)
                    )
                return a

            return lax.fori_loop(0, S // unroll, body, init)

        scale = jnp.float32(1.0 / S)
        zero = jnp.zeros((16,), jnp.float32)

        def do_row(r, n, do_fire):
            drain(n)
            accs = accum(bufsa.at[n], (zero,) * nvec)
            accs = accum(bufsb.at[n], accs)
            if do_fire:
                fire(r + _NR, n)
            for q in range(nvec):
                out_v[r, pl.ds(16 * q, 16)] = accs[q] * scale

        for n in range(_NR):
            fire(n, n)

        def loop_body(g, _):
            for n in range(_NR):
                do_row(_NR * g + n, n, True)
            return 0

        lax.fori_loop(0, rows_per_w // _NR - 1, loop_body, 0)
        for n in range(_NR):
            do_row(rows_per_w - _NR + n, n, False)

        pltpu.sync_copy(out_v, out_hbm.at[pl.ds(rbase, rows_per_w)])

    return k(xa, xb, table, fixup)


@jax.jit
def _embed_mean(x, embedding_weight):
    V, E = embedding_weight.shape
    v_main = (V // 128) * 128
    pairs = _transpose_table(embedding_weight.T, v_main)
    table = pairs.reshape(v_main, E)
    fixup = jnp.concatenate(
        [
            jnp.zeros((1, E), jnp.float32),
            embedding_weight[v_main:] - embedding_weight[v_main - 1],
        ],
        axis=0,
    )
    xa = _remap(jnp.minimum(x, v_main - 1))
    xb = jnp.maximum(x - (v_main - 1), 0)
    return _gather_mean(xa, xb, table, fixup)


def kernel(x, embedding_weight):
    return _embed_mean(x, embedding_weight)


# speedup vs baseline: 36.0509x; 23.5711x over previous
import functools

import jax
import jax.numpy as jnp
from jax import lax
from jax.experimental import pallas as pl
from jax.experimental.pallas import tpu as pltpu
from jax.experimental.pallas import tpu_sc as plsc

_NR = 4


def _gather_mean(x, table):
    B, S = x.shape
    V, E = table.shape
    NC, NS = 2, 16
    NW = NC * NS
    rows_per_w = B // NW
    nvec = E // 16
    s_a = 128
    s_b = S - s_a
    mesh = plsc.VectorSubcoreMesh(core_axis_name="c", subcore_axis_name="s")

    @functools.partial(
        pl.kernel,
        mesh=mesh,
        out_type=jax.ShapeDtypeStruct((B, E), jnp.float32),
        compiler_params=pltpu.CompilerParams(use_tc_tiling_on_sc=False),
        scratch_types=[
            pltpu.VMEM((rows_per_w, S), jnp.int32),
            pltpu.VMEM((_NR, S, E), jnp.float32),
            pltpu.VMEM((rows_per_w, E), jnp.float32),
        ]
        + [pltpu.SemaphoreType.DMA] * _NR,
    )
    def k(x_hbm, tab_hbm, out_hbm, idx_v, bufs, out_v, *sems):
        wid = lax.axis_index("s") * NC + lax.axis_index("c")
        rbase = wid * rows_per_w
        pltpu.sync_copy(x_hbm.at[pl.ds(rbase, rows_per_w)], idx_v)

        def fire(r, n):
            pltpu.make_async_copy(
                tab_hbm.at[idx_v.at[r, pl.ds(0, s_a)]],
                bufs.at[n, pl.ds(0, s_a)],
                sems[n],
            ).start()
            pltpu.make_async_copy(
                tab_hbm.at[idx_v.at[r, pl.ds(s_a, s_b)]],
                bufs.at[n, pl.ds(s_a, s_b)],
                sems[n],
            ).start()

        def drain(n):
            pltpu.make_async_copy(
                tab_hbm.at[idx_v.at[0, pl.ds(0, s_a)]],
                bufs.at[n, pl.ds(0, s_a)],
                sems[n],
            ).wait()
            pltpu.make_async_copy(
                tab_hbm.at[idx_v.at[0, pl.ds(s_a, s_b)]],
                bufs.at[n, pl.ds(s_a, s_b)],
                sems[n],
            ).wait()

        def accum(n):
            buf = bufs.at[n]
            unroll = 8
            zero = jnp.zeros((16,), jnp.float32)

            def body(t, a):
                base = t * unroll
                for i in range(unroll):
                    a = tuple(
                        a[q] + buf[base + i, pl.ds(16 * q, 16)]
                        for q in range(nvec)
                    )
                return a

            return lax.fori_loop(0, S // unroll, body, (zero,) * nvec)

        scale = jnp.float32(1.0 / S)

        def do_row(r, n, do_fire):
            drain(n)
            accs = accum(n)
            if do_fire:
                fire(r + _NR, n)
            for q in range(nvec):
                out_v[r, pl.ds(16 * q, 16)] = accs[q] * scale

        for n in range(_NR):
            fire(n, n)

        def loop_body(g, _):
            for n in range(_NR):
                do_row(_NR * g + n, n, True)
            return 0

        lax.fori_loop(0, rows_per_w // _NR - 1, loop_body, 0)
        for n in range(_NR):
            do_row(rows_per_w - _NR + n, n, False)

        pltpu.sync_copy(out_v, out_hbm.at[pl.ds(rbase, rows_per_w)])

    return k(x, table)


def kernel(x, embedding_weight):
    return _gather_mean(x, embedding_weight)
